# baseline (speedup 1.0000x reference)
import functools

import jax
import jax.numpy as jnp
from jax import lax
from jax.experimental import pallas as pl
from jax.experimental.pallas import tpu as pltpu
from jax.experimental.pallas import tpu_sc as plsc

BATCH = 16384
EMBED_DIM = 64
VOCAB = 1000000
NUM_CORES = 2
NUM_SUBCORES = 16
NUM_WORKERS = NUM_CORES * NUM_SUBCORES
LANES = 16

GROUP = 128
NGROUPS = (VOCAB + GROUP - 1) // GROUP
TAIL_GROUP = VOCAB // GROUP
NGW = 246
PTR_LEN = 7936
OUT_ROWS = BATCH + NUM_WORKERS
OUT_W = 128

_mesh = plsc.VectorSubcoreMesh(core_axis_name="c", subcore_axis_name="s")


@functools.partial(
    pl.kernel,
    out_type=jax.ShapeDtypeStruct((OUT_ROWS, OUT_W), jnp.float32),
    mesh=_mesh,
    compiler_params=pltpu.CompilerParams(use_tc_tiling_on_sc=True,
                                         needs_layout_passes=False),
    scratch_types=[
        pltpu.VMEM((PTR_LEN,), jnp.int32),
        pltpu.VMEM((BATCH,), jnp.int32),
        pltpu.VMEM((BATCH,), jnp.int32),
        pltpu.VMEM((EMBED_DIM, GROUP), jnp.float32),
        pltpu.VMEM((EMBED_DIM, GROUP), jnp.float32),
        pltpu.VMEM((EMBED_DIM, GROUP), jnp.float32),
        pltpu.VMEM((EMBED_DIM, GROUP), jnp.float32),
        pltpu.VMEM((LANES, OUT_W), jnp.float32),
        pltpu.VMEM((LANES,), jnp.int32),
        pltpu.SemaphoreType.DMA,
        pltpu.SemaphoreType.DMA,
        pltpu.SemaphoreType.DMA,
    ],
)
def _encoder_kernel(ptr_hbm, sidx_hbm, perm_hbm, item_hbm, ent_hbm,
                    tail_item_hbm, tail_ent_hbm, out_hbm,
                    ptr_v, sidx_v, perm_v, buf_a0, buf_b0, buf_a1, buf_b1,
                    obuf, pbuf, sem_i, sem_0, sem_1):
    wid = lax.axis_index("s") * NUM_CORES + lax.axis_index("c")
    dump_row = BATCH + wid
    pltpu.async_copy(ptr_hbm, ptr_v, sem_i).wait()
    pltpu.async_copy(sidx_hbm, sidx_v, sem_i).wait()
    pltpu.async_copy(perm_hbm, perm_v, sem_i).wait()
    iota = lax.iota(jnp.int32, LANES)

    def scalars(j):
        g = wid * NGW + j
        win = ptr_v[pl.ds(g, LANES)]
        return g, win[0], win[1]

    def descriptors(g, ba, bb, sem):
        goff = jnp.minimum(g, TAIL_GROUP - 1) * GROUP
        full = [
            pltpu.make_async_copy(item_hbm.at[:, pl.ds(goff, GROUP)], ba, sem),
            pltpu.make_async_copy(ent_hbm.at[:, pl.ds(goff, GROUP)], bb, sem),
        ]
        tail = [
            pltpu.make_async_copy(tail_item_hbm, ba.at[:, pl.ds(0, 128)], sem),
            pltpu.make_async_copy(tail_ent_hbm, bb.at[:, pl.ds(0, 128)], sem),
        ]
        return full, tail

    def prefetch(g, s, e, ba, bb, sem):
        @pl.when(s < e)
        def _():
            full, tail = descriptors(g, ba, bb, sem)

            @pl.when(g < TAIL_GROUP)
            def _():
                for d in full:
                    d.start()

            @pl.when(g == TAIL_GROUP)
            def _():
                for d in tail:
                    d.start()

    def wait_bufs(g, s, e, ba, bb, sem):
        @pl.when(s < e)
        def _():
            full, tail = descriptors(g, ba, bb, sem)

            @pl.when(g < TAIL_GROUP)
            def _():
                for d in full:
                    d.wait()

            @pl.when(g == TAIL_GROUP)
            def _():
                for d in tail:
                    d.wait()

    def process(g, s, e, ba, bb):
        @pl.when(s < e)
        def _():
            goff = g * GROUP
            lofs = jnp.where(g == TAIL_GROUP, 64, 0).astype(jnp.int32)
            s_al = s & ~(LANES - 1)

            @pl.loop(s_al, e, step=LANES)
            def _chunk(k):
                kvec = k + iota
                mask = jnp.logical_and(kvec >= s, kvec < e)
                svec = sidx_v[pl.ds(k, LANES)]
                pvec = perm_v[pl.ds(k, LANES)]
                lvec = jnp.bitwise_and(svec - goff + lofs, GROUP - 1)
                pbuf[...] = jnp.where(mask, pvec, dump_row)
                for d in range(EMBED_DIM):
                    dvec = jnp.full((LANES,), d, jnp.int32)
                    va = plsc.load_gather(ba, [dvec, lvec])
                    vb = plsc.load_gather(bb, [dvec, lvec])
                    plsc.store_scatter(obuf, [iota, dvec], va + vb)
                pltpu.sync_copy(obuf, out_hbm.at[pbuf])

    g0, s0, e0 = scalars(0)
    prefetch(g0, s0, e0, buf_a0, buf_b0, sem_0)

    @pl.loop(0, NGW, step=2)
    def _pair(j):
        g_a, s_a, e_a = scalars(j)
        g_b, s_b, e_b = scalars(j + 1)
        prefetch(g_b, s_b, e_b, buf_a1, buf_b1, sem_1)
        wait_bufs(g_a, s_a, e_a, buf_a0, buf_b0, sem_0)
        process(g_a, s_a, e_a, buf_a0, buf_b0)
        g_c, s_c, e_c = scalars(j + 2)
        in_range = (j + 2 < NGW).astype(jnp.int32)
        prefetch(g_c, s_c * in_range, e_c * in_range, buf_a0, buf_b0, sem_0)
        wait_bufs(g_b, s_b, e_b, buf_a1, buf_b1, sem_1)
        process(g_b, s_b, e_b, buf_a1, buf_b1)


def kernel(batch_data, item_table, ent_table):
    idx32 = batch_data.astype(jnp.int32)
    pos = jnp.arange(BATCH, dtype=jnp.int32)
    sidx, perm = jax.lax.sort((idx32, pos), num_keys=1)
    bounds = jnp.arange(NGROUPS + 1, dtype=jnp.int32) * GROUP
    ptr = jnp.searchsorted(sidx, bounds, side="left",
                           method="sort").astype(jnp.int32)
    ptr = jnp.concatenate(
        [ptr, jnp.full((PTR_LEN - NGROUPS - 1,), BATCH, jnp.int32)])
    full = _encoder_kernel(ptr, sidx, perm, item_table.T, ent_table.T,
                           item_table[VOCAB - 128:].T,
                           ent_table[VOCAB - 128:].T)
    return full[:BATCH, :EMBED_DIM]


# speedup vs baseline: 1.2318x; 1.2318x over previous
import functools

import jax
import jax.numpy as jnp
from jax import lax
from jax.experimental import pallas as pl
from jax.experimental.pallas import tpu as pltpu
from jax.experimental.pallas import tpu_sc as plsc

BATCH = 16384
EMBED_DIM = 64
VOCAB = 1000000
NUM_CORES = 2
NUM_SUBCORES = 16
NUM_WORKERS = NUM_CORES * NUM_SUBCORES
LANES = 16

GROUP = 256
NGROUPS = (VOCAB + GROUP - 1) // GROUP
TAIL_GROUP = VOCAB // GROUP
NGW = 124
PTR_LEN = 4032
OUT_ROWS = BATCH + NUM_WORKERS
OUT_W = 128

_mesh = plsc.VectorSubcoreMesh(core_axis_name="c", subcore_axis_name="s")


@functools.partial(
    pl.kernel,
    out_type=jax.ShapeDtypeStruct((OUT_ROWS, OUT_W), jnp.float32),
    mesh=_mesh,
    compiler_params=pltpu.CompilerParams(use_tc_tiling_on_sc=True,
                                         needs_layout_passes=False),
    scratch_types=[
        pltpu.VMEM((PTR_LEN,), jnp.int32),
        pltpu.VMEM((BATCH,), jnp.int32),
        pltpu.VMEM((BATCH,), jnp.int32),
        pltpu.VMEM((EMBED_DIM, GROUP), jnp.float32),
        pltpu.VMEM((EMBED_DIM, GROUP), jnp.float32),
        pltpu.VMEM((EMBED_DIM, GROUP), jnp.float32),
        pltpu.VMEM((EMBED_DIM, GROUP), jnp.float32),
        pltpu.VMEM((LANES, OUT_W), jnp.float32),
        pltpu.VMEM((LANES,), jnp.int32),
        pltpu.SemaphoreType.DMA,
        pltpu.SemaphoreType.DMA,
        pltpu.SemaphoreType.DMA,
    ],
)
def _encoder_kernel(ptr_hbm, sidx_hbm, perm_hbm, item_hbm, ent_hbm,
                    tail_item_hbm, tail_ent_hbm, out_hbm,
                    ptr_v, sidx_v, perm_v, buf_a0, buf_b0, buf_a1, buf_b1,
                    obuf, pbuf, sem_i, sem_0, sem_1):
    wid = lax.axis_index("s") * NUM_CORES + lax.axis_index("c")
    dump_row = BATCH + wid
    pltpu.async_copy(ptr_hbm, ptr_v, sem_i).wait()
    pltpu.async_copy(sidx_hbm, sidx_v, sem_i).wait()
    pltpu.async_copy(perm_hbm, perm_v, sem_i).wait()
    iota = lax.iota(jnp.int32, LANES)

    def scalars(j):
        g = wid * NGW + j
        win = ptr_v[pl.ds(g, LANES)]
        return g, win[0], win[1]

    def descriptors(g, ba, bb, sem):
        goff = jnp.minimum(g, TAIL_GROUP - 1) * GROUP
        full = [
            pltpu.make_async_copy(item_hbm.at[:, pl.ds(goff, GROUP)], ba, sem),
            pltpu.make_async_copy(ent_hbm.at[:, pl.ds(goff, GROUP)], bb, sem),
        ]
        tail = [
            pltpu.make_async_copy(tail_item_hbm, ba.at[:, pl.ds(0, 128)], sem),
            pltpu.make_async_copy(tail_ent_hbm, bb.at[:, pl.ds(0, 128)], sem),
        ]
        return full, tail

    def prefetch(g, s, e, ba, bb, sem):
        @pl.when(s < e)
        def _():
            full, tail = descriptors(g, ba, bb, sem)

            @pl.when(g < TAIL_GROUP)
            def _():
                for d in full:
                    d.start()

            @pl.when(g == TAIL_GROUP)
            def _():
                for d in tail:
                    d.start()

    def wait_bufs(g, s, e, ba, bb, sem):
        @pl.when(s < e)
        def _():
            full, tail = descriptors(g, ba, bb, sem)

            @pl.when(g < TAIL_GROUP)
            def _():
                for d in full:
                    d.wait()

            @pl.when(g == TAIL_GROUP)
            def _():
                for d in tail:
                    d.wait()

    def process(g, s, e, ba, bb):
        @pl.when(s < e)
        def _():
            goff = g * GROUP
            lofs = jnp.where(g == TAIL_GROUP, 64, 0).astype(jnp.int32)
            s_al = s & ~(LANES - 1)

            @pl.loop(s_al, e, step=LANES)
            def _chunk(k):
                kvec = k + iota
                mask = jnp.logical_and(kvec >= s, kvec < e)
                svec = sidx_v[pl.ds(k, LANES)]
                pvec = perm_v[pl.ds(k, LANES)]
                lvec = jnp.bitwise_and(svec - goff + lofs, GROUP - 1)
                pbuf[...] = jnp.where(mask, pvec, dump_row)
                for d in range(EMBED_DIM):
                    dvec = jnp.full((LANES,), d, jnp.int32)
                    va = plsc.load_gather(ba, [dvec, lvec])
                    vb = plsc.load_gather(bb, [dvec, lvec])
                    plsc.store_scatter(obuf, [iota, dvec], va + vb)
                pltpu.sync_copy(obuf, out_hbm.at[pbuf])

    g0, s0, e0 = scalars(0)
    prefetch(g0, s0, e0, buf_a0, buf_b0, sem_0)

    @pl.loop(0, NGW, step=2)
    def _pair(j):
        g_a, s_a, e_a = scalars(j)
        g_b, s_b, e_b = scalars(j + 1)
        prefetch(g_b, s_b, e_b, buf_a1, buf_b1, sem_1)
        wait_bufs(g_a, s_a, e_a, buf_a0, buf_b0, sem_0)
        process(g_a, s_a, e_a, buf_a0, buf_b0)
        g_c, s_c, e_c = scalars(j + 2)
        in_range = (j + 2 < NGW).astype(jnp.int32)
        prefetch(g_c, s_c * in_range, e_c * in_range, buf_a0, buf_b0, sem_0)
        wait_bufs(g_b, s_b, e_b, buf_a1, buf_b1, sem_1)
        process(g_b, s_b, e_b, buf_a1, buf_b1)


def kernel(batch_data, item_table, ent_table):
    idx32 = batch_data.astype(jnp.int32)
    pos = jnp.arange(BATCH, dtype=jnp.int32)
    sidx, perm = jax.lax.sort((idx32, pos), num_keys=1)
    bounds = jnp.arange(NGROUPS + 1, dtype=jnp.int32) * GROUP
    ptr = jnp.searchsorted(sidx, bounds, side="left",
                           method="sort").astype(jnp.int32)
    ptr = jnp.concatenate(
        [ptr, jnp.full((PTR_LEN - NGROUPS - 1,), BATCH, jnp.int32)])
    full = _encoder_kernel(ptr, sidx, perm, item_table.T, ent_table.T,
                           item_table[VOCAB - 128:].T,
                           ent_table[VOCAB - 128:].T)
    return full[:BATCH, :EMBED_DIM]


# speedup vs baseline: 1.4597x; 1.1850x over previous
import functools

import jax
import jax.numpy as jnp
from jax import lax
from jax.experimental import pallas as pl
from jax.experimental.pallas import tpu as pltpu
from jax.experimental.pallas import tpu_sc as plsc

BATCH = 16384
EMBED_DIM = 64
VOCAB = 1000000
NUM_CORES = 2
NUM_SUBCORES = 16
NUM_WORKERS = NUM_CORES * NUM_SUBCORES
LANES = 16

GROUP = 256
NGROUPS = (VOCAB + GROUP - 1) // GROUP
TAIL_GROUP = VOCAB // GROUP
NGW = 124
PTR_LEN = 4032
OUT_ROWS = BATCH + NUM_WORKERS
OUT_W = 128

_mesh = plsc.VectorSubcoreMesh(core_axis_name="c", subcore_axis_name="s")


@functools.partial(
    pl.kernel,
    out_type=jax.ShapeDtypeStruct((OUT_ROWS, OUT_W), jnp.float32),
    mesh=_mesh,
    compiler_params=pltpu.CompilerParams(use_tc_tiling_on_sc=True,
                                         needs_layout_passes=False),
    scratch_types=[
        pltpu.VMEM((PTR_LEN,), jnp.int32),
        pltpu.VMEM((BATCH,), jnp.int32),
        pltpu.VMEM((BATCH,), jnp.int32),
        pltpu.VMEM((EMBED_DIM, GROUP), jnp.float32),
        pltpu.VMEM((EMBED_DIM, GROUP), jnp.float32),
        pltpu.VMEM((EMBED_DIM, GROUP), jnp.float32),
        pltpu.VMEM((EMBED_DIM, GROUP), jnp.float32),
        pltpu.VMEM((LANES, OUT_W), jnp.float32),
        pltpu.VMEM((LANES,), jnp.int32),
        pltpu.SemaphoreType.DMA,
        pltpu.SemaphoreType.DMA,
        pltpu.SemaphoreType.DMA,
    ],
)
def _encoder_kernel(ptr_hbm, sidx_hbm, perm_hbm, item_hbm, ent_hbm,
                    tail_item_hbm, tail_ent_hbm, out_hbm,
                    ptr_v, sidx_v, perm_v, buf_a0, buf_b0, buf_a1, buf_b1,
                    obuf, pbuf, sem_i, sem_0, sem_1):
    wid = lax.axis_index("s") * NUM_CORES + lax.axis_index("c")
    dump_row = BATCH + wid
    pltpu.async_copy(ptr_hbm, ptr_v, sem_i).wait()
    pltpu.async_copy(sidx_hbm, sidx_v, sem_i).wait()
    pltpu.async_copy(perm_hbm, perm_v, sem_i).wait()
    iota = lax.iota(jnp.int32, LANES)

    def scalars(j):
        g = wid * NGW + j
        win = ptr_v[pl.ds(g, LANES)]
        return g, win[0], win[1]

    def descriptors(g, ba, bb, sem):
        goff = jnp.minimum(g, TAIL_GROUP - 1) * GROUP
        full = [
            pltpu.make_async_copy(item_hbm.at[:, pl.ds(goff, GROUP)], ba, sem),
            pltpu.make_async_copy(ent_hbm.at[:, pl.ds(goff, GROUP)], bb, sem),
        ]
        tail = [
            pltpu.make_async_copy(tail_item_hbm, ba.at[:, pl.ds(0, 128)], sem),
            pltpu.make_async_copy(tail_ent_hbm, bb.at[:, pl.ds(0, 128)], sem),
        ]
        return full, tail

    def prefetch(g, s, e, ba, bb, sem):
        @pl.when(s < e)
        def _():
            full, tail = descriptors(g, ba, bb, sem)

            @pl.when(g < TAIL_GROUP)
            def _():
                for d in full:
                    d.start()

            @pl.when(g == TAIL_GROUP)
            def _():
                for d in tail:
                    d.start()

    def wait_bufs(g, s, e, ba, bb, sem):
        @pl.when(s < e)
        def _():
            full, tail = descriptors(g, ba, bb, sem)

            @pl.when(g < TAIL_GROUP)
            def _():
                for d in full:
                    d.wait()

            @pl.when(g == TAIL_GROUP)
            def _():
                for d in tail:
                    d.wait()

    def process(g, s, e, ba, bb):
        @pl.when(s < e)
        def _():
            goff = g * GROUP
            lofs = jnp.where(g == TAIL_GROUP, 64, 0).astype(jnp.int32)
            s_al = s & ~(LANES - 1)

            @pl.loop(s_al, e, step=LANES)
            def _chunk(k):
                kvec = k + iota
                mask = jnp.logical_and(kvec >= s, kvec < e)
                svec = sidx_v[pl.ds(k, LANES)]
                pvec = perm_v[pl.ds(k, LANES)]
                lvec = jnp.bitwise_and(svec - goff + lofs, GROUP - 1)
                pbuf[...] = jnp.where(mask, pvec, dump_row)
                for d in range(EMBED_DIM):
                    dvec = jnp.full((LANES,), d, jnp.int32)
                    va = plsc.load_gather(ba, [dvec, lvec])
                    vb = plsc.load_gather(bb, [dvec, lvec])
                    plsc.store_scatter(obuf, [iota, dvec], va + vb)
                pltpu.sync_copy(obuf, out_hbm.at[pbuf])

    g0, s0, e0 = scalars(0)
    prefetch(g0, s0, e0, buf_a0, buf_b0, sem_0)

    @pl.loop(0, NGW, step=2)
    def _pair(j):
        g_a, s_a, e_a = scalars(j)
        g_b, s_b, e_b = scalars(j + 1)
        prefetch(g_b, s_b, e_b, buf_a1, buf_b1, sem_1)
        wait_bufs(g_a, s_a, e_a, buf_a0, buf_b0, sem_0)
        process(g_a, s_a, e_a, buf_a0, buf_b0)
        g_c, s_c, e_c = scalars(j + 2)
        in_range = (j + 2 < NGW).astype(jnp.int32)
        prefetch(g_c, s_c * in_range, e_c * in_range, buf_a0, buf_b0, sem_0)
        wait_bufs(g_b, s_b, e_b, buf_a1, buf_b1, sem_1)
        process(g_b, s_b, e_b, buf_a1, buf_b1)


def kernel(batch_data, item_table, ent_table):
    idx32 = batch_data.astype(jnp.int32)
    pos = jnp.arange(BATCH, dtype=jnp.int32)
    sidx, perm = jax.lax.sort((idx32, pos), num_keys=1)
    bounds = jnp.arange(NGROUPS + 1, dtype=jnp.int32) * GROUP
    ptr = jnp.searchsorted(sidx, bounds, side="left",
                           method="compare_all").astype(jnp.int32)
    ptr = jnp.concatenate(
        [ptr, jnp.full((PTR_LEN - NGROUPS - 1,), BATCH, jnp.int32)])
    full = _encoder_kernel(ptr, sidx, perm, item_table.T, ent_table.T,
                           item_table[VOCAB - 128:].T,
                           ent_table[VOCAB - 128:].T)
    return full[:BATCH, :EMBED_DIM]
